# Initial kernel scaffold; baseline (speedup 1.0000x reference)
import jax
import jax.numpy as jnp
from jax.experimental import pallas as pl


def kernel(xyz, params):
    raise NotImplementedError("write your pallas kernel here")


# speedup vs baseline: 11.9444x; 11.9444x over previous
import functools

import jax
import jax.numpy as jnp
from jax import lax
from jax.experimental import pallas as pl
from jax.experimental.pallas import tpu as pltpu
from jax.experimental.pallas import tpu_sc as plsc

B = 8
N = 4096
NPT = 512
KNN = 32
M = B * NPT * KNN
MT = 2048
NW = 32
RPW = M // NW
ICHUNK = 128
NCH = RPW // ICHUNK
F32 = jnp.float32
HIGH = lax.Precision.HIGHEST


def _fps_body(xyz_ref, far_ref, nxyz_ref, dist_ref):
    dist_ref[...] = jnp.full((B, N), 1e10, F32)
    lane = lax.broadcasted_iota(jnp.int32, (B, N), 1)
    pcol = lax.broadcasted_iota(jnp.int32, (B, NPT), 1)
    x0, x1, x2 = xyz_ref[0], xyz_ref[1], xyz_ref[2]

    def body(i, far):
        ohf = (lane == far).astype(F32)
        c0 = jnp.sum(x0 * ohf, axis=1, keepdims=True)
        c1 = jnp.sum(x1 * ohf, axis=1, keepdims=True)
        c2 = jnp.sum(x2 * ohf, axis=1, keepdims=True)
        onp = pcol == i
        nxyz_ref[0] = jnp.where(onp, c0, nxyz_ref[0])
        nxyz_ref[1] = jnp.where(onp, c1, nxyz_ref[1])
        nxyz_ref[2] = jnp.where(onp, c2, nxyz_ref[2])
        d = (x0 - c0) ** 2
        d = d + (x1 - c1) ** 2
        d = d + (x2 - c2) ** 2
        dist = jnp.minimum(dist_ref[...], d)
        dist_ref[...] = dist
        m = jnp.max(dist, axis=1, keepdims=True)
        return jnp.min(jnp.where(dist == m, lane, N), axis=1, keepdims=True)

    lax.fori_loop(0, NPT, body, far_ref[...])


_fps_call = pl.pallas_call(
    _fps_body,
    out_shape=jax.ShapeDtypeStruct((3, B, NPT), F32),
    scratch_shapes=[pltpu.VMEM((B, N), F32)],
)


def _knn_body(s_ref, d_ref, idx_ref, dist_ref):
    b = pl.program_id(0)
    S = s_ref[0]
    Dt = d_ref[0]
    G = lax.dot_general(S, Dt, (((1,), (0,)), ((), ())),
                        preferred_element_type=F32)
    s2 = jnp.sum(S * S, axis=1, keepdims=True)
    d2 = jnp.sum(Dt * Dt, axis=0, keepdims=True)
    dist_ref[...] = (-2.0 * G + s2) + d2
    lane = lax.broadcasted_iota(jnp.int32, (NPT, N), 1)
    kcol = lax.broadcasted_iota(jnp.int32, (NPT, KNN), 1)

    def it(k, acc):
        dist = dist_ref[...]
        m = jnp.min(dist, axis=1, keepdims=True)
        sel = jnp.min(jnp.where(dist == m, lane, N), axis=1, keepdims=True)
        dist_ref[...] = jnp.where(lane == sel, jnp.float32(jnp.inf), dist)
        return jnp.where(kcol == k, sel, acc)

    acc = lax.fori_loop(0, KNN, it, jnp.zeros((NPT, KNN), jnp.int32))
    idx_ref[0] = acc + b * N


_knn_call = pl.pallas_call(
    _knn_body,
    grid=(B,),
    in_specs=[
        pl.BlockSpec((1, NPT, 4), lambda b: (b, 0, 0)),
        pl.BlockSpec((1, 4, N), lambda b: (b, 0, 0)),
    ],
    out_specs=pl.BlockSpec((1, NPT, KNN), lambda b: (b, 0, 0)),
    out_shape=jax.ShapeDtypeStruct((B, NPT, KNN), jnp.int32),
    scratch_shapes=[pltpu.VMEM((NPT, N), F32)],
)


@functools.partial(
    pl.kernel,
    out_type=jax.ShapeDtypeStruct((3 * M,), F32),
    mesh=plsc.VectorSubcoreMesh(core_axis_name="c", subcore_axis_name="s"),
    compiler_params=pltpu.CompilerParams(needs_layout_passes=False),
    scratch_types=[
        pltpu.VMEM((RPW,), jnp.int32),
        pltpu.VMEM((N,), F32),
        pltpu.VMEM((N,), F32),
        pltpu.VMEM((N,), F32),
        pltpu.VMEM((RPW,), F32),
        pltpu.VMEM((RPW,), F32),
        pltpu.VMEM((RPW,), F32),
    ],
)
def _sc_gather(planes_hbm, idx_hbm, out_hbm,
               idx_v, t0, t1, t2, r0, r1, r2):
    wid = lax.axis_index("s") * 2 + lax.axis_index("c")
    batch = wid // (NW // B)
    pltpu.sync_copy(idx_hbm.at[pl.ds(wid * RPW, RPW)], idx_v)
    pltpu.sync_copy(planes_hbm.at[pl.ds(batch * N, N)], t0)
    pltpu.sync_copy(planes_hbm.at[pl.ds(B * N + batch * N, N)], t1)
    pltpu.sync_copy(planes_hbm.at[pl.ds(2 * B * N + batch * N, N)], t2)

    def step(i, _):
        iv = idx_v[pl.ds(i * 16, 16)] - batch * N
        r0[pl.ds(i * 16, 16)] = plsc.load_gather(t0, [iv])
        r1[pl.ds(i * 16, 16)] = plsc.load_gather(t1, [iv])
        r2[pl.ds(i * 16, 16)] = plsc.load_gather(t2, [iv])
        return 0

    lax.fori_loop(0, RPW // 16, step, 0)
    pltpu.sync_copy(r0, out_hbm.at[pl.ds(wid * RPW, RPW)])
    pltpu.sync_copy(r1, out_hbm.at[pl.ds(M + wid * RPW, RPW)])
    pltpu.sync_copy(r2, out_hbm.at[pl.ds(2 * M + wid * RPW, RPW)])


def _sums_update(t, y, s_ref):
    cur = jnp.concatenate(
        [jnp.sum(y, axis=0, keepdims=True),
         jnp.sum(y * y, axis=0, keepdims=True)], axis=0)

    @pl.when(t == 0)
    def _():
        s_ref[...] = cur

    @pl.when(t != 0)
    def _():
        s_ref[...] = s_ref[...] + cur


def _l1_body(x_ref, c_ref, w_ref, b_ref, y_ref, s_ref):
    t = pl.program_id(0)
    X = x_ref[...]
    C = c_ref[...]
    Cexp = jnp.broadcast_to(C[:, None, :], (MT // KNN, KNN, 4)).reshape(MT, 4)
    Xn = X - Cexp
    Y = lax.dot_general(Xn, w_ref[...], (((1,), (1,)), ((), ())),
                        preferred_element_type=F32)
    Y = Y + b_ref[...]
    y_ref[...] = Y
    _sums_update(t, Y, s_ref)


def _mid_body(y_ref, sc_ref, sh_ref, w_ref, b_ref, o_ref, s_ref):
    t = pl.program_id(0)
    Z = jnp.maximum(y_ref[...] * sc_ref[...] + sh_ref[...], 0.0)
    Y = lax.dot_general(Z, w_ref[...], (((1,), (1,)), ((), ())),
                        preferred_element_type=F32)
    Y = Y + b_ref[...]
    o_ref[...] = Y
    _sums_update(t, Y, s_ref)


def _last_body(y_ref, sc_ref, sh_ref, w_ref, b_ref, o_ref, s_ref):
    t = pl.program_id(0)
    Z = jnp.maximum(y_ref[...] * sc_ref[...] + sh_ref[...], 0.0)
    Y = lax.dot_general(Z, w_ref[...], (((1,), (1,)), ((), ())),
                        preferred_element_type=F32)
    Y = Y + b_ref[...]
    _sums_update(t, Y, s_ref)
    o_ref[...] = jnp.max(Y.reshape(MT // KNN, KNN, 256), axis=1)


def _fin_body(y_ref, sc_ref, sh_ref, o_ref):
    o_ref[...] = jnp.maximum(y_ref[...] * sc_ref[...] + sh_ref[...], 0.0)


def _row_spec(c):
    return pl.BlockSpec((MT, c), lambda t: (t, 0))


def _const_spec(shape):
    nd = len(shape)
    return pl.BlockSpec(shape, lambda t: (0,) * nd)


def _make_layer(body, cin, cout, extra_in, out_rows):
    ins = [_row_spec(cin)] + extra_in + [
        _const_spec((cout, cin)), _const_spec((1, cout))]
    return pl.pallas_call(
        body,
        grid=(M // MT,),
        in_specs=ins,
        out_specs=[
            pl.BlockSpec((out_rows, cout), lambda t: (t, 0)),
            _const_spec((2, cout)),
        ],
        out_shape=[
            jax.ShapeDtypeStruct((M // MT * out_rows, cout), F32),
            jax.ShapeDtypeStruct((2, cout), F32),
        ],
    )


_l1_call = _make_layer(
    _l1_body, 4, 64, [pl.BlockSpec((MT // KNN, 4), lambda t: (t, 0))], MT)
_l2_call = _make_layer(
    _mid_body, 64, 128, [_const_spec((1, 64)), _const_spec((1, 64))], MT)
_l3_call = _make_layer(
    _last_body, 128, 256, [_const_spec((1, 128)), _const_spec((1, 128))],
    MT // KNN)

_fin_call = pl.pallas_call(
    _fin_body,
    grid=(2,),
    in_specs=[
        pl.BlockSpec((MT, 256), lambda t: (t, 0)),
        _const_spec((1, 256)),
        _const_spec((1, 256)),
    ],
    out_specs=pl.BlockSpec((MT, 256), lambda t: (t, 0)),
    out_shape=jax.ShapeDtypeStruct((B * NPT, 256), F32),
)


def _bn_coeffs(sums, gamma, beta):
    mean = sums[0] / M
    var = sums[1] / M - mean * mean
    scale = gamma / jnp.sqrt(var + 1e-5)
    shift = beta - mean * scale
    return scale.reshape(1, -1), shift.reshape(1, -1)


def kernel(xyz, params):
    (W1, b1, g1, be1, W2, b2, g2, be2, W3, b3, g3, be3) = params

    xyz_t = jnp.transpose(xyz, (2, 0, 1))
    far0 = jax.random.randint(jax.random.key(42), (B,), 0, N)
    far0 = far0.astype(jnp.int32).reshape(B, 1)
    nxyz3 = _fps_call(xyz_t, far0)
    new_xyz = jnp.transpose(nxyz3, (1, 2, 0))

    xyz_tp = jnp.concatenate(
        [jnp.transpose(xyz_t, (1, 0, 2)), jnp.zeros((B, 1, N), F32)], axis=1)
    nx_p = jnp.concatenate([new_xyz, jnp.zeros((B, NPT, 1), F32)], axis=-1)

    idx = _knn_call(nx_p, xyz_tp)
    planes = xyz_t.reshape(3 * B * N)
    Xp = _sc_gather(planes, idx.reshape(M)).reshape(3, M)
    X = jnp.concatenate(
        [jnp.transpose(Xp, (1, 0)), jnp.zeros((M, 1), F32)], axis=-1)

    W1p = jnp.concatenate([W1, jnp.zeros((64, 1), F32)], axis=1)
    nx_flat = nx_p.reshape(B * NPT, 4)

    Y1, sums1 = _l1_call(X, nx_flat, W1p, b1.reshape(1, -1))
    sc1, sh1 = _bn_coeffs(sums1, g1, be1)
    Y2, sums2 = _l2_call(Y1, sc1, sh1, W2, b2.reshape(1, -1))
    sc2, sh2 = _bn_coeffs(sums2, g2, be2)
    Y3m, sums3 = _l3_call(Y2, sc2, sh2, W3, b3.reshape(1, -1))
    sc3, sh3 = _bn_coeffs(sums3, g3, be3)
    outp = _fin_call(Y3m, sc3, sh3)

    new_points = jnp.transpose(outp.reshape(B, NPT, 256), (0, 2, 1))
    return (new_xyz, new_points)
